# Initial kernel scaffold; baseline (speedup 1.0000x reference)
import jax
import jax.numpy as jnp
from jax.experimental import pallas as pl


def kernel(x, edge_index, batch, W1, att_src1, att_dst1, b1, W2, att_src2, att_dst2, b2, W3, att_src3, att_dst3, b3):
    raise NotImplementedError("write your pallas kernel here")


# speedup vs baseline: 24.9900x; 24.9900x over previous
import functools

import jax
import jax.numpy as jnp
from jax import lax
from jax.experimental import pallas as pl
from jax.experimental.pallas import tpu as pltpu, tpu_sc as plsc

N_CORES = 2
N_SUB = 16
N_W = N_CORES * N_SUB
CH = 80


def _mesh():
    return plsc.VectorSubcoreMesh(
        core_axis_name="c", subcore_axis_name="s",
        num_cores=N_CORES, num_subcores=N_SUB)


def _sc_gather(table, idx):
    r, d = table.shape
    b = idx.shape[0]
    per_w = b // N_W
    n_it = per_w // CH

    @functools.partial(
        pl.kernel, mesh=_mesh(),
        out_type=jax.ShapeDtypeStruct((b, d), jnp.float32),
        scratch_types=[
            pltpu.VMEM((CH,), jnp.int32),
            pltpu.VMEM((CH, d), jnp.float32),
            pltpu.SemaphoreType.DMA,
        ],
    )
    def k(tbl, ix, out, ixv, rows, sem):
        wid = lax.axis_index("s") * N_CORES + lax.axis_index("c")
        base = wid * per_w

        def body(i, carry):
            off = base + i * CH
            pltpu.sync_copy(ix.at[pl.ds(off, CH)], ixv)
            pltpu.async_copy(tbl.at[ixv], rows, sem).wait()
            pltpu.sync_copy(rows, out.at[pl.ds(off, CH)])
            return carry

        lax.fori_loop(0, n_it, body, 0)

    return k(table, idx)


def _sc_scatter_add(vals, idx, zeros, col):
    b = vals.shape[0]
    t = zeros.shape[0]
    per_w = b // N_W
    n_it = per_w // CH

    @functools.partial(
        pl.kernel, mesh=_mesh(),
        out_type=jax.ShapeDtypeStruct((N_CORES, t, 128), jnp.float32),
        scratch_types=[
            pltpu.VMEM((CH,), jnp.int32),
            pltpu.VMEM((CH, 128), jnp.float32),
            pltpu.VMEM_SHARED((t, 128), jnp.float32),
        ],
    )
    def k(v, ix, zer, out, ixv, valv, shared):
        c = lax.axis_index("c")
        s = lax.axis_index("s")
        wid = s * N_CORES + c
        base = wid * per_w

        @pl.when(s == 0)
        def _init():
            pltpu.sync_copy(zer, shared)

        plsc.subcore_barrier()

        def body(i, carry):
            off = base + i * CH
            pltpu.sync_copy(ix.at[pl.ds(off, CH)], ixv)
            pltpu.sync_copy(v.at[pl.ds(off, CH), pl.ds(col, 128)], valv)
            pltpu.sync_copy(valv, shared.at[ixv], add=True)
            return carry

        lax.fori_loop(0, n_it, body, 0)
        plsc.subcore_barrier()

        @pl.when(s == 0)
        def _flush():
            pltpu.sync_copy(shared, out.at[c])

    return k(vals, idx, zeros)


def _node_prep(x, w, ablk_s, ablk_d):
    n = x.shape[0]
    hc = w.shape[1]
    dw = (hc + 8 + 127) // 128 * 128
    pad = dw - hc - 8

    bn = 2000
    grid = n // bn

    def body(x_ref, w_ref, as_ref, ad_ref, s_ref, dp_ref, m_ref):
        i = pl.program_id(0)
        xp = jnp.dot(x_ref[...], w_ref[...], preferred_element_type=jnp.float32)
        a_s = jnp.dot(xp, as_ref[...], preferred_element_type=jnp.float32)
        a_d = jnp.dot(xp, ad_ref[...], preferred_element_type=jnp.float32)
        z = jnp.zeros((xp.shape[0], pad), jnp.float32)
        zd = jnp.zeros((xp.shape[0], 120), jnp.float32)
        s_ref[...] = jnp.concatenate([xp, a_s, z], axis=1)
        dp_ref[...] = jnp.concatenate([a_d, zd], axis=1)
        m = (jnp.max(a_s, axis=0, keepdims=True)
             + jnp.max(a_d, axis=0, keepdims=True))
        m = jnp.concatenate([m, jnp.zeros_like(m)], axis=1)

        @pl.when(i == 0)
        def _():
            m_ref[...] = m

        @pl.when(i > 0)
        def _():
            m_ref[...] = jnp.maximum(m_ref[...], m)

    return pl.pallas_call(
        body,
        grid=(grid,),
        in_specs=[
            pl.BlockSpec((bn, x.shape[1]), lambda i: (i, 0)),
            pl.BlockSpec(w.shape, lambda i: (0, 0)),
            pl.BlockSpec(ablk_s.shape, lambda i: (0, 0)),
            pl.BlockSpec(ablk_d.shape, lambda i: (0, 0)),
        ],
        out_specs=(
            pl.BlockSpec((bn, dw), lambda i: (i, 0)),
            pl.BlockSpec((bn, 128), lambda i: (i, 0)),
            pl.BlockSpec((1, 16), lambda i: (0, 0)),
        ),
        out_shape=(
            jax.ShapeDtypeStruct((n, dw), jnp.float32),
            jax.ShapeDtypeStruct((n, 128), jnp.float32),
            jax.ShapeDtypeStruct((1, 16), jnp.float32),
        ),
    )(x, w, ablk_s, ablk_d)


def _edge_compute(g1, de, m, b8, hc):
    e = g1.shape[0]
    dw = (hc + 8 + 127) // 128 * 128
    pad = dw - hc - 8
    be = 2000
    grid = e // be

    def body(g1_ref, de_ref, m_ref, b8_ref, out_ref):
        g1b = g1_ref[...]
        gx = g1b[:, :hc]
        ae = g1b[:, hc:hc + 8]
        t = ae + de_ref[:, :8]
        t = jnp.where(t >= 0.0, t, 0.2 * t)
        mv = m_ref[:, :8]
        mlr = jnp.where(mv >= 0.0, mv, 0.2 * mv)
        w = jnp.exp(t - mlr)
        wf = jnp.dot(w, b8_ref[...], preferred_element_type=jnp.float32)
        z = jnp.zeros((gx.shape[0], pad), jnp.float32)
        out_ref[...] = jnp.concatenate([gx * wf, w, z], axis=1)

    return pl.pallas_call(
        body,
        grid=(grid,),
        in_specs=[
            pl.BlockSpec((be, dw), lambda i: (i, 0)),
            pl.BlockSpec((be, 128), lambda i: (i, 0)),
            pl.BlockSpec((1, 16), lambda i: (0, 0)),
            pl.BlockSpec((8, hc), lambda i: (0, 0)),
        ],
        out_specs=pl.BlockSpec((be, dw), lambda i: (i, 0)),
        out_shape=jax.ShapeDtypeStruct((e, dw), jnp.float32),
    )(g1, de, m, b8)


def _finalize12(pa, pb, b8, bias, x_res):
    n = pa.shape[1]
    args = [pa[0], pa[1], pb[0], pb[1], b8, bias] + (
        [] if x_res is None else [x_res])

    bn = 2000
    grid = n // bn

    def body(*refs):
        out_ref = refs[-1]
        num = refs[0][...] + refs[1][...]
        wacc = refs[2][...] + refs[3][...]
        den = wacc[:, :8]
        denb = jnp.dot(den, refs[4][...], preferred_element_type=jnp.float32)
        o = num / (denb + 1e-16) + refs[5][...]
        if x_res is not None:
            o = o + refs[6][...]
        oneg = jnp.minimum(o, 0.0)
        out_ref[...] = jnp.where(o > 0.0, o, jnp.exp(oneg) - 1.0)

    row = lambda i: (i, 0)
    const = lambda i: (0, 0)
    in_specs = [pl.BlockSpec((bn, 128), row)] * 4 + [
        pl.BlockSpec((8, 128), const), pl.BlockSpec((1, 128), const)]
    if x_res is not None:
        in_specs.append(pl.BlockSpec((bn, 128), row))
    return pl.pallas_call(
        body,
        grid=(grid,),
        in_specs=in_specs,
        out_specs=pl.BlockSpec((bn, 128), row),
        out_shape=jax.ShapeDtypeStruct((n, 128), jnp.float32),
    )(*args)


def _finalize3(pa, pb, pc, rep, bias):
    n = pa.shape[1]

    bn = 2000
    grid = n // bn

    def body(a0, a1, b0, b1, c0, c1, rep_ref, bias_ref, out_ref):
        acc_a = a0[...] + a1[...]
        acc_b = b0[...] + b1[...]
        acc_c = c0[...] + c1[...]
        num = jnp.concatenate([acc_a, acc_b, acc_c[:, :64]], axis=1)
        den = acc_c[:, 64:72]
        repm = rep_ref[...]
        denb = jnp.dot(den, repm, preferred_element_type=jnp.float32)
        o = num / (denb + 1e-16)
        s = jnp.zeros((o.shape[0], 40), jnp.float32)
        for h in range(8):
            s = s + o[:, h * 40:(h + 1) * 40]
        out_ref[...] = s * 0.125 + bias_ref[...]

    row = lambda i: (i, 0)
    const = lambda i: (0, 0)
    return pl.pallas_call(
        body,
        grid=(grid,),
        in_specs=[pl.BlockSpec((bn, 128), row)] * 6 + [
            pl.BlockSpec((8, 320), const), pl.BlockSpec((1, 40), const)],
        out_specs=pl.BlockSpec((bn, 40), row),
        out_shape=jax.ShapeDtypeStruct((n, 40), jnp.float32),
    )(pa[0], pa[1], pb[0], pb[1], pc[0], pc[1], rep, bias)


def _blockdiag(att):
    h, c = att.shape
    return (att[:, :, None] * jnp.eye(h, dtype=att.dtype)[:, None, :]).reshape(h * c, h)


def _gat_layer(x, src, dst, w, att_src, att_dst, hc):
    s_tbl, d_tbl, m = _node_prep(x, w, _blockdiag(att_src), _blockdiag(att_dst))
    g1 = _sc_gather(s_tbl, src)
    de = _sc_gather(d_tbl, dst)
    b8 = jnp.repeat(jnp.eye(8, dtype=jnp.float32), hc // 8, axis=1)
    msg = _edge_compute(g1, de, m, b8, hc)
    return msg, b8


def kernel(x, edge_index, batch, W1, att_src1, att_dst1, b1,
           W2, att_src2, att_dst2, b2, W3, att_src3, att_dst3, b3):
    n = x.shape[0]
    src = edge_index[0].astype(jnp.int32)
    dst = edge_index[1].astype(jnp.int32)
    z = jnp.zeros((n, 128), jnp.float32)

    msg, b8 = _gat_layer(x, src, dst, W1, att_src1, att_dst1, 128)
    pa = _sc_scatter_add(msg, dst, z, 0)
    pb = _sc_scatter_add(msg, dst, z, 128)
    x1 = _finalize12(pa, pb, b8, b1.reshape(1, 128), None)

    msg, _ = _gat_layer(x1, src, dst, W2, att_src2, att_dst2, 128)
    pa = _sc_scatter_add(msg, dst, z, 0)
    pb = _sc_scatter_add(msg, dst, z, 128)
    x2 = _finalize12(pa, pb, b8, b2.reshape(1, 128), x1)

    msg, _ = _gat_layer(x2, src, dst, W3, att_src3, att_dst3, 320)
    pa = _sc_scatter_add(msg, dst, z, 0)
    pb = _sc_scatter_add(msg, dst, z, 128)
    pc = _sc_scatter_add(msg, dst, z, 256)
    rep = jnp.repeat(jnp.eye(8, dtype=jnp.float32), 40, axis=1)
    return _finalize3(pa, pb, pc, rep, b3.reshape(1, 40))
